# baseline (speedup 1.0000x reference)
import functools

import jax
import jax.numpy as jnp
from jax import lax
from jax.experimental import pallas as pl
from jax.experimental.pallas import tpu as pltpu
from jax.experimental.pallas import tpu_sc as plsc

EMB = 128
NF = 26
NFEAT = 4
NG = EMB // 16
NW = 32
G = 2


def _sc_encoder(srcf, uniqf, am, lut, sbias, ubias, bsz):
    cb = bsz // NW
    nch = cb // G
    nq = nch // 4
    spb = NF * NFEAT
    spc = G * spb
    upp = 2 * G * NF

    mesh = plsc.VectorSubcoreMesh(core_axis_name="c", subcore_axis_name="s")

    @functools.partial(
        pl.kernel,
        out_type=[
            jax.ShapeDtypeStruct((bsz, EMB), jnp.float32),
            jax.ShapeDtypeStruct((bsz, NF, EMB), jnp.float32),
            jax.ShapeDtypeStruct((bsz, EMB), jnp.float32),
        ],
        mesh=mesh,
        compiler_params=pltpu.CompilerParams(use_tc_tiling_on_sc=True),
        scratch_types=[
            pltpu.VMEM((cb * spb,), jnp.int32),
            pltpu.VMEM((cb * NF,), jnp.int32),
            pltpu.VMEM((cb * NF + 16,), jnp.float32),
            pltpu.VMEM((spc, EMB), jnp.float32),
            pltpu.VMEM((spc, EMB), jnp.float32),
            pltpu.VMEM((upp, EMB), jnp.float32),
            pltpu.VMEM((upp, EMB), jnp.float32),
            pltpu.VMEM((G, NF, EMB), jnp.float32),
            pltpu.VMEM((G, NF, EMB), jnp.float32),
            pltpu.VMEM((2 * G, EMB), jnp.float32),
            pltpu.VMEM((2 * G, EMB), jnp.float32),
            pltpu.VMEM((2 * G, EMB), jnp.float32),
            pltpu.VMEM((2 * G, EMB), jnp.float32),
            pltpu.VMEM((EMB,), jnp.float32),
            pltpu.VMEM((EMB,), jnp.float32),
            pltpu.SemaphoreType.DMA,
            pltpu.SemaphoreType.DMA,
            pltpu.SemaphoreType.DMA,
            pltpu.SemaphoreType.DMA,
            pltpu.SemaphoreType.DMA,
            pltpu.SemaphoreType.DMA,
        ],
    )
    def k(src_h, uniq_h, am_h, lut_h, sb_h, ub_h,
          senc_h, sfe_h, uq_h,
          sidx_v, uidx_v, am_v, srows0, srows1, urowsA, urowsB,
          sfe0, sfe1, sencA, sencB, uqA, uqB, sb_v, ub_v,
          gsem0, gsem1, usemA, usemB, fsem, psem):
        wid = lax.axis_index("s") * 2 + lax.axis_index("c")
        srows = (srows0, srows1)
        gsem = (gsem0, gsem1)
        urows = (urowsA, urowsB)
        usem = (usemA, usemB)
        sfe = (sfe0, sfe1)
        senc = (sencA, sencB)
        uq = (uqA, uqB)
        pltpu.sync_copy(sb_h, sb_v)
        pltpu.sync_copy(ub_h, ub_v)
        pltpu.sync_copy(src_h.at[pl.ds(wid * cb * spb, cb * spb)], sidx_v)
        pltpu.sync_copy(uniq_h.at[pl.ds(wid * cb * NF, cb * NF)], uidx_v)
        pltpu.sync_copy(am_h.at[pl.ds(wid * cb * NF, cb * NF)],
                        am_v.at[pl.ds(0, cb * NF)])
        zero = jnp.zeros((16,), jnp.float32)
        sbr = [sb_v[pl.ds(g * 16, 16)] for g in range(NG)]
        ubr = [ub_v[pl.ds(g * 16, 16)] for g in range(NG)]

        def gcopy(c, p):
            return [pltpu.make_async_copy(
                lut_h.at[sidx_v.at[pl.ds((c * G + b) * spb, spb)]],
                srows[p].at[pl.ds(b * spb, spb)], gsem[p])
                for b in range(G)]

        def ucopy(pr, q):
            return [pltpu.make_async_copy(
                lut_h.at[uidx_v.at[pl.ds(pr * upp, upp)]],
                urows[q], usem[q])]

        def fcopy(c, p):
            return [pltpu.make_async_copy(
                sfe[p], sfe_h.at[pl.ds(wid * cb + c * G, G)], fsem)]

        def pcopy(pr, q):
            base = wid * cb + pr * 2 * G
            return [
                pltpu.make_async_copy(senc[q], senc_h.at[pl.ds(base, 2 * G)],
                                      psem),
                pltpu.make_async_copy(uq[q], uq_h.at[pl.ds(base, 2 * G)],
                                      psem),
            ]

        def compute(c, p, q, half):
            for b in range(G):
                def fbody(f, macc):
                    amw = am_v[pl.ds((c * G + b) * NF + f, 16)]
                    am_s = jnp.full((16,), amw[0], jnp.float32)
                    r0 = b * spb + f * NFEAT
                    rows = [[srows[p][r0 + kk, pl.ds(g * 16, 16)]
                             for kk in range(NFEAT)] for g in range(NG)]
                    e = []
                    for g in range(NG):
                        s = (rows[g][0] + rows[g][1]) + (rows[g][2] + rows[g][3])
                        e.append(jnp.maximum(s + sbr[g], 0.0))
                    for g in range(NG):
                        sfe[p][b, f, pl.ds(g * 16, 16)] = e[g]
                    return tuple(jnp.maximum(macc[g], e[g] * am_s)
                                 for g in range(NG))

                macc = lax.fori_loop(0, NF, fbody, (zero,) * NG)
                for g in range(NG):
                    senc[q][half * G + b, pl.ds(g * 16, 16)] = macc[g]

                def ubody(f2, acc):
                    r = (half * G + b) * NF + f2 * 2
                    l0 = [urows[q][r, pl.ds(g * 16, 16)] for g in range(NG)]
                    l1 = [urows[q][r + 1, pl.ds(g * 16, 16)]
                          for g in range(NG)]
                    return tuple(acc[g] + (l0[g] + l1[g]) for g in range(NG))

                uacc = lax.fori_loop(0, NF // 2, ubody, (zero,) * NG)
                for g in range(NG):
                    sl = pl.ds(g * 16, 16)
                    uq[q][half * G + b, sl] = jnp.maximum(uacc[g] + ubr[g],
                                                          0.0)

        def start(descs):
            for d in descs:
                d.start()

        def wait(descs):
            for d in descs:
                d.wait()

        start(gcopy(0, 0))
        start(gcopy(1, 1))
        start(ucopy(0, 0))
        start(ucopy(1, 1))

        def step(s2, carry):
            c0 = s2 * 4
            pr = s2 * 2

            @pl.when(s2 > 0)
            def _():
                wait(fcopy(c0 - 2, 0))
                wait(pcopy(pr - 2, 0))

            wait(gcopy(c0, 0))
            wait(ucopy(pr, 0))
            compute(c0, 0, 0, 0)
            start(gcopy(c0 + 2, 0))
            start(fcopy(c0, 0))

            @pl.when(s2 > 0)
            def _():
                wait(fcopy(c0 - 1, 1))

            wait(gcopy(c0 + 1, 1))
            compute(c0 + 1, 1, 0, 1)
            start(gcopy(c0 + 3, 1))
            start(fcopy(c0 + 1, 1))
            start(pcopy(pr, 0))

            @pl.when(s2 < nq - 1)
            def _():
                start(ucopy(pr + 2, 0))

            @pl.when(s2 > 0)
            def _():
                wait(pcopy(pr - 1, 1))

            wait(fcopy(c0, 0))
            wait(gcopy(c0 + 2, 0))
            wait(ucopy(pr + 1, 1))
            compute(c0 + 2, 0, 1, 0)

            @pl.when(s2 < nq - 1)
            def _():
                start(gcopy(c0 + 4, 0))

            start(fcopy(c0 + 2, 0))

            wait(fcopy(c0 + 1, 1))
            wait(gcopy(c0 + 3, 1))
            compute(c0 + 3, 1, 1, 1)

            @pl.when(s2 < nq - 1)
            def _():
                start(gcopy(c0 + 5, 1))
                start(ucopy(pr + 3, 1))

            start(fcopy(c0 + 3, 1))
            start(pcopy(pr + 1, 1))
            return carry

        lax.fori_loop(0, nq, step, 0)
        wait(fcopy(nch - 2, 0))
        wait(fcopy(nch - 1, 1))
        wait(pcopy(2 * nq - 2, 0))
        wait(pcopy(2 * nq - 1, 1))

    return k(srcf, uniqf, am, lut, sbias, ubias)


def kernel(src, avgmask, uniqfields, lut, src_bias, uniq_bias):
    bsz, nf, _ = src.shape
    srcf = src.reshape(-1).astype(jnp.int32)
    uniqf = uniqfields.reshape(-1).astype(jnp.int32)
    senc, sfe, uenc = _sc_encoder(srcf, uniqf, avgmask.reshape(-1), lut,
                                  src_bias.reshape(-1), uniq_bias.reshape(-1),
                                  bsz)
    return senc, sfe, uenc


# speedup vs baseline: 1.1796x; 1.1796x over previous
import functools

import jax
import jax.numpy as jnp
from jax import lax
from jax.experimental import pallas as pl
from jax.experimental.pallas import tpu as pltpu
from jax.experimental.pallas import tpu_sc as plsc

EMB = 128
NF = 26
NFEAT = 4
NG = EMB // 16
NW = 32
G = 2


def _sc_encoder(srcf, uniqf, am, lut, sbias, ubias, bsz):
    cb = bsz // NW
    nch = cb // G
    nq = nch // 4
    spb = NF * NFEAT
    spc = G * spb
    upp = 2 * G * NF

    mesh = plsc.VectorSubcoreMesh(core_axis_name="c", subcore_axis_name="s")

    @functools.partial(
        pl.kernel,
        out_type=[
            jax.ShapeDtypeStruct((bsz, EMB), jnp.float32),
            jax.ShapeDtypeStruct((NF, bsz, EMB), jnp.float32),
            jax.ShapeDtypeStruct((bsz, EMB), jnp.float32),
        ],
        mesh=mesh,
        compiler_params=pltpu.CompilerParams(use_tc_tiling_on_sc=True),
        scratch_types=[
            pltpu.VMEM((cb * spb,), jnp.int32),
            pltpu.VMEM((cb * NF,), jnp.int32),
            pltpu.VMEM((cb * NF + 16,), jnp.float32),
            pltpu.VMEM((spc, EMB), jnp.float32),
            pltpu.VMEM((spc, EMB), jnp.float32),
            pltpu.VMEM((upp, EMB), jnp.float32),
            pltpu.VMEM((upp, EMB), jnp.float32),
            pltpu.VMEM((NF, 4 * G, EMB), jnp.float32),
            pltpu.VMEM((2 * G, EMB), jnp.float32),
            pltpu.VMEM((2 * G, EMB), jnp.float32),
            pltpu.VMEM((2 * G, EMB), jnp.float32),
            pltpu.VMEM((2 * G, EMB), jnp.float32),
            pltpu.VMEM((EMB,), jnp.float32),
            pltpu.VMEM((EMB,), jnp.float32),
            pltpu.SemaphoreType.DMA,
            pltpu.SemaphoreType.DMA,
            pltpu.SemaphoreType.DMA,
            pltpu.SemaphoreType.DMA,
            pltpu.SemaphoreType.DMA,
            pltpu.SemaphoreType.DMA,
        ],
    )
    def k(src_h, uniq_h, am_h, lut_h, sb_h, ub_h,
          senc_h, sfe_h, uq_h,
          sidx_v, uidx_v, am_v, srows0, srows1, urowsA, urowsB,
          sfe_v, sencA, sencB, uqA, uqB, sb_v, ub_v,
          gsem0, gsem1, usemA, usemB, fsem, psem):
        wid = lax.axis_index("s") * 2 + lax.axis_index("c")
        srows = (srows0, srows1)
        gsem = (gsem0, gsem1)
        urows = (urowsA, urowsB)
        usem = (usemA, usemB)
        senc = (sencA, sencB)
        uq = (uqA, uqB)
        pltpu.sync_copy(sb_h, sb_v)
        pltpu.sync_copy(ub_h, ub_v)
        pltpu.sync_copy(src_h.at[pl.ds(wid * cb * spb, cb * spb)], sidx_v)
        pltpu.sync_copy(uniq_h.at[pl.ds(wid * cb * NF, cb * NF)], uidx_v)
        pltpu.sync_copy(am_h.at[pl.ds(wid * cb * NF, cb * NF)],
                        am_v.at[pl.ds(0, cb * NF)])
        zero = jnp.zeros((16,), jnp.float32)
        sbr = [sb_v[pl.ds(g * 16, 16)] for g in range(NG)]
        ubr = [ub_v[pl.ds(g * 16, 16)] for g in range(NG)]

        def gcopy(c, p):
            return [pltpu.make_async_copy(
                lut_h.at[sidx_v.at[pl.ds((c * G + b) * spb, spb)]],
                srows[p].at[pl.ds(b * spb, spb)], gsem[p])
                for b in range(G)]

        def ucopy(pr, q):
            return [pltpu.make_async_copy(
                lut_h.at[uidx_v.at[pl.ds(pr * upp, upp)]],
                urows[q], usem[q])]

        def fcopy(s2):
            return [pltpu.make_async_copy(
                sfe_v, sfe_h.at[:, pl.ds(wid * cb + s2 * 4 * G, 4 * G)],
                fsem)]

        def pcopy(pr, q):
            base = wid * cb + pr * 2 * G
            return [
                pltpu.make_async_copy(senc[q], senc_h.at[pl.ds(base, 2 * G)],
                                      psem),
                pltpu.make_async_copy(uq[q], uq_h.at[pl.ds(base, 2 * G)],
                                      psem),
            ]

        def compute(c, p, q, half, j):
            for b in range(G):
                def fbody(f, macc):
                    amw = am_v[pl.ds((c * G + b) * NF + f, 16)]
                    am_s = jnp.full((16,), amw[0], jnp.float32)
                    r0 = b * spb + f * NFEAT
                    rows = [[srows[p][r0 + kk, pl.ds(g * 16, 16)]
                             for kk in range(NFEAT)] for g in range(NG)]
                    e = []
                    for g in range(NG):
                        s = (rows[g][0] + rows[g][1]) + (rows[g][2] + rows[g][3])
                        e.append(jnp.maximum(s + sbr[g], 0.0))
                    for g in range(NG):
                        sfe_v[f, j * G + b, pl.ds(g * 16, 16)] = e[g]
                    return tuple(jnp.maximum(macc[g], e[g] * am_s)
                                 for g in range(NG))

                macc = lax.fori_loop(0, NF, fbody, (zero,) * NG)
                for g in range(NG):
                    senc[q][half * G + b, pl.ds(g * 16, 16)] = macc[g]

                def ubody(f2, acc):
                    r = (half * G + b) * NF + f2 * 2
                    l0 = [urows[q][r, pl.ds(g * 16, 16)] for g in range(NG)]
                    l1 = [urows[q][r + 1, pl.ds(g * 16, 16)]
                          for g in range(NG)]
                    return tuple(acc[g] + (l0[g] + l1[g]) for g in range(NG))

                uacc = lax.fori_loop(0, NF // 2, ubody, (zero,) * NG)
                for g in range(NG):
                    sl = pl.ds(g * 16, 16)
                    uq[q][half * G + b, sl] = jnp.maximum(uacc[g] + ubr[g],
                                                          0.0)

        def start(descs):
            for d in descs:
                d.start()

        def wait(descs):
            for d in descs:
                d.wait()

        start(gcopy(0, 0))
        start(gcopy(1, 1))
        start(ucopy(0, 0))
        start(ucopy(1, 1))

        def step(s2, carry):
            c0 = s2 * 4
            pr = s2 * 2

            @pl.when(s2 > 0)
            def _():
                wait(fcopy(s2 - 1))
                wait(pcopy(pr - 2, 0))

            wait(gcopy(c0, 0))
            wait(ucopy(pr, 0))
            compute(c0, 0, 0, 0, 0)
            start(gcopy(c0 + 2, 0))

            wait(gcopy(c0 + 1, 1))
            compute(c0 + 1, 1, 0, 1, 1)
            start(gcopy(c0 + 3, 1))
            start(pcopy(pr, 0))

            @pl.when(s2 < nq - 1)
            def _():
                start(ucopy(pr + 2, 0))

            @pl.when(s2 > 0)
            def _():
                wait(pcopy(pr - 1, 1))

            wait(gcopy(c0 + 2, 0))
            wait(ucopy(pr + 1, 1))
            compute(c0 + 2, 0, 1, 0, 2)

            @pl.when(s2 < nq - 1)
            def _():
                start(gcopy(c0 + 4, 0))

            wait(gcopy(c0 + 3, 1))
            compute(c0 + 3, 1, 1, 1, 3)

            @pl.when(s2 < nq - 1)
            def _():
                start(gcopy(c0 + 5, 1))
                start(ucopy(pr + 3, 1))

            start(fcopy(s2))
            start(pcopy(pr + 1, 1))
            return carry

        lax.fori_loop(0, nq, step, 0)
        wait(fcopy(nq - 1))
        wait(pcopy(2 * nq - 2, 0))
        wait(pcopy(2 * nq - 1, 1))

    return k(srcf, uniqf, am, lut, sbias, ubias)


def kernel(src, avgmask, uniqfields, lut, src_bias, uniq_bias):
    bsz, nf, _ = src.shape
    srcf = src.reshape(-1).astype(jnp.int32)
    uniqf = uniqfields.reshape(-1).astype(jnp.int32)
    senc, sfeT, uenc = _sc_encoder(srcf, uniqf, avgmask.reshape(-1), lut,
                                   src_bias.reshape(-1), uniq_bias.reshape(-1),
                                   bsz)
    return senc, sfeT.transpose(1, 0, 2), uenc


# speedup vs baseline: 1.5657x; 1.3274x over previous
import functools

import jax
import jax.numpy as jnp
from jax import lax
from jax.experimental import pallas as pl
from jax.experimental.pallas import tpu as pltpu
from jax.experimental.pallas import tpu_sc as plsc

EMB = 128
NF = 26
NFEAT = 4
NG = EMB // 16
NW = 32
G = 2


def _sc_encoder(srcf, uniqf, am, lut, sbias, ubias, bsz):
    cb = bsz // NW
    nch = cb // G
    nq = nch // 4
    spb = NF * NFEAT
    spc = G * spb
    upp = 2 * G * NF

    mesh = plsc.VectorSubcoreMesh(core_axis_name="c", subcore_axis_name="s")

    @functools.partial(
        pl.kernel,
        out_type=[
            jax.ShapeDtypeStruct((bsz, EMB), jnp.float32),
            jax.ShapeDtypeStruct((NF, bsz, EMB), jnp.float32),
            jax.ShapeDtypeStruct((bsz, EMB), jnp.float32),
        ],
        mesh=mesh,
        compiler_params=pltpu.CompilerParams(use_tc_tiling_on_sc=True),
        scratch_types=[
            pltpu.VMEM((cb * spb,), jnp.int32),
            pltpu.VMEM((cb * NF,), jnp.int32),
            pltpu.VMEM((cb * NF + 16,), jnp.float32),
            pltpu.VMEM((spc, EMB), jnp.float32),
            pltpu.VMEM((spc, EMB), jnp.float32),
            pltpu.VMEM((upp, EMB), jnp.float32),
            pltpu.VMEM((upp, EMB), jnp.float32),
            pltpu.VMEM((NF, 4 * G, EMB), jnp.float32),
            pltpu.VMEM((2 * G, EMB), jnp.float32),
            pltpu.VMEM((2 * G, EMB), jnp.float32),
            pltpu.VMEM((2 * G, EMB), jnp.float32),
            pltpu.VMEM((2 * G, EMB), jnp.float32),
            pltpu.VMEM((EMB,), jnp.float32),
            pltpu.VMEM((EMB,), jnp.float32),
            pltpu.SemaphoreType.DMA,
            pltpu.SemaphoreType.DMA,
            pltpu.SemaphoreType.DMA,
            pltpu.SemaphoreType.DMA,
            pltpu.SemaphoreType.DMA,
            pltpu.SemaphoreType.DMA,
        ],
    )
    def k(src_h, uniq_h, am_h, lut_h, sb_h, ub_h,
          senc_h, sfe_h, uq_h,
          sidx_v, uidx_v, am_v, srows0, srows1, urowsA, urowsB,
          sfe_v, sencA, sencB, uqA, uqB, sb_v, ub_v,
          gsem0, gsem1, usemA, usemB, fsem, psem):
        wid = lax.axis_index("s") * 2 + lax.axis_index("c")
        srows = (srows0, srows1)
        gsem = (gsem0, gsem1)
        urows = (urowsA, urowsB)
        usem = (usemA, usemB)
        senc = (sencA, sencB)
        uq = (uqA, uqB)
        pltpu.sync_copy(sb_h, sb_v)
        pltpu.sync_copy(ub_h, ub_v)
        pltpu.sync_copy(src_h.at[pl.ds(wid * cb * spb, cb * spb)], sidx_v)
        pltpu.sync_copy(uniq_h.at[pl.ds(wid * cb * NF, cb * NF)], uidx_v)
        pltpu.sync_copy(am_h.at[pl.ds(wid * cb * NF, cb * NF)],
                        am_v.at[pl.ds(0, cb * NF)])
        zero = jnp.zeros((16,), jnp.float32)
        sbr = [sb_v[pl.ds(g * 16, 16)] for g in range(NG)]
        ubr = [ub_v[pl.ds(g * 16, 16)] for g in range(NG)]

        def gcopy(c, p):
            return [pltpu.make_async_copy(
                lut_h.at[sidx_v.at[pl.ds((c * G + b) * spb, spb)]],
                srows[p].at[pl.ds(b * spb, spb)], gsem[p])
                for b in range(G)]

        def ucopy(pr, q):
            return [pltpu.make_async_copy(
                lut_h.at[uidx_v.at[pl.ds(pr * upp, upp)]],
                urows[q], usem[q])]

        def fcopy(s2):
            return [pltpu.make_async_copy(
                sfe_v, sfe_h.at[:, pl.ds(wid * cb + s2 * 4 * G, 4 * G)],
                fsem)]

        def pcopy(pr, q):
            base = wid * cb + pr * 2 * G
            return [
                pltpu.make_async_copy(senc[q], senc_h.at[pl.ds(base, 2 * G)],
                                      psem),
                pltpu.make_async_copy(uq[q], uq_h.at[pl.ds(base, 2 * G)],
                                      psem),
            ]

        def compute(c, p, q, half, j):
            for b in range(G):
                def fbody(f, macc):
                    amw = am_v[pl.ds((c * G + b) * NF + f, 16)]
                    am_s = jnp.full((16,), amw[0], jnp.float32)
                    r0 = b * spb + f * NFEAT
                    rows = [[srows[p][r0 + kk, pl.ds(g * 16, 16)]
                             for kk in range(NFEAT)] for g in range(NG)]
                    e = []
                    for g in range(NG):
                        s = (rows[g][0] + rows[g][1]) + (rows[g][2] + rows[g][3])
                        e.append(jnp.maximum(s + sbr[g], 0.0))
                    for g in range(NG):
                        sfe_v[f, j * G + b, pl.ds(g * 16, 16)] = e[g]
                    return tuple(jnp.maximum(macc[g], e[g] * am_s)
                                 for g in range(NG))

                macc = lax.fori_loop(0, NF, fbody, (zero,) * NG)
                for g in range(NG):
                    senc[q][half * G + b, pl.ds(g * 16, 16)] = macc[g]

                def ubody(f2, acc):
                    r = (half * G + b) * NF + f2 * 2
                    l0 = [urows[q][r, pl.ds(g * 16, 16)] for g in range(NG)]
                    l1 = [urows[q][r + 1, pl.ds(g * 16, 16)]
                          for g in range(NG)]
                    return tuple(acc[g] + (l0[g] + l1[g]) for g in range(NG))

                uacc = lax.fori_loop(0, NF // 2, ubody, (zero,) * NG)
                for g in range(NG):
                    sl = pl.ds(g * 16, 16)
                    uq[q][half * G + b, sl] = jnp.maximum(uacc[g] + ubr[g],
                                                          0.0)

        def start(descs):
            for d in descs:
                d.start()

        def wait(descs):
            for d in descs:
                d.wait()

        start(gcopy(0, 0))
        start(gcopy(1, 1))
        start(ucopy(0, 0))
        start(ucopy(1, 1))

        def step(s2, carry):
            c0 = s2 * 4
            pr = s2 * 2

            @pl.when(s2 > 0)
            def _():
                wait(fcopy(s2 - 1))
                wait(pcopy(pr - 2, 0))

            wait(gcopy(c0, 0))
            wait(ucopy(pr, 0))
            compute(c0, 0, 0, 0, 0)
            start(gcopy(c0 + 2, 0))

            wait(gcopy(c0 + 1, 1))
            compute(c0 + 1, 1, 0, 1, 1)
            start(gcopy(c0 + 3, 1))
            start(pcopy(pr, 0))

            @pl.when(s2 < nq - 1)
            def _():
                start(ucopy(pr + 2, 0))

            @pl.when(s2 > 0)
            def _():
                wait(pcopy(pr - 1, 1))

            wait(gcopy(c0 + 2, 0))
            wait(ucopy(pr + 1, 1))
            compute(c0 + 2, 0, 1, 0, 2)

            @pl.when(s2 < nq - 1)
            def _():
                start(gcopy(c0 + 4, 0))

            wait(gcopy(c0 + 3, 1))
            compute(c0 + 3, 1, 1, 1, 3)

            @pl.when(s2 < nq - 1)
            def _():
                start(gcopy(c0 + 5, 1))
                start(ucopy(pr + 3, 1))

            start(fcopy(s2))
            start(pcopy(pr + 1, 1))
            return carry

        lax.fori_loop(0, nq, step, 0)
        wait(fcopy(nq - 1))
        wait(pcopy(2 * nq - 2, 0))
        wait(pcopy(2 * nq - 1, 1))

    return k(srcf, uniqf, am, lut, sbias, ubias)


def _tc_transpose(x, bsz):
    spb = NF * NFEAT
    blk = bsz // 8

    def body(x_ref, o_ref):
        o_ref[...] = x_ref[...].T

    return pl.pallas_call(
        body,
        grid=(8,),
        in_specs=[pl.BlockSpec((spb, blk), lambda i: (0, i))],
        out_specs=pl.BlockSpec((blk, spb), lambda i: (i, 0)),
        out_shape=jax.ShapeDtypeStruct((bsz, spb), jnp.int32),
    )(x)


def kernel(src, avgmask, uniqfields, lut, src_bias, uniq_bias):
    bsz, nf, _ = src.shape
    srcf = _tc_transpose(
        src.transpose(1, 2, 0).reshape(nf * NFEAT, bsz).astype(jnp.int32),
        bsz).reshape(-1)
    uniqf = uniqfields.reshape(-1).astype(jnp.int32)
    senc, sfeT, uenc = _sc_encoder(srcf, uniqf, avgmask.reshape(-1), lut,
                                   src_bias.reshape(-1), uniq_bias.reshape(-1),
                                   bsz)
    return senc, sfeT.transpose(1, 0, 2), uenc
